# baseline (speedup 1.0000x reference)
import functools

import jax
import jax.numpy as jnp
from jax import lax
from jax.experimental import pallas as pl
from jax.experimental.pallas import tpu as pltpu

H = 128
ATOM_D = 128
BOND_D = 144
N_ATOMS = 10001
N_BONDS = 320000
MAX_NB = 32
N_MOLS = 100
MOL_SIZE = 100
DEPTH = 3

NA_PAD = 10240
NB_PAD = 327680


def _pad_rows(x, n):
    return jnp.pad(x, ((0, n - x.shape[0]),) + ((0, 0),) * (x.ndim - 1))


def _mm(xws, adds=(), bias=None, relu=False, pre=None, block=512, rows=None):
    n = xws[0][0].shape[0]
    rows = n if rows is None else rows
    grid = rows // block
    hout = xws[0][1].shape[1]
    in_specs = []
    args = []
    for x, w in xws:
        in_specs.append(pl.BlockSpec((block, x.shape[1]), lambda i: (i, 0)))
        in_specs.append(pl.BlockSpec(w.shape, lambda i: (0, 0)))
        args += [x, w]
    for a in adds:
        in_specs.append(pl.BlockSpec((block, a.shape[1]), lambda i: (i, 0)))
        args.append(a)
    if bias is not None:
        b2 = bias.reshape(1, hout)
        in_specs.append(pl.BlockSpec((1, hout), lambda i: (0, 0)))
        args.append(b2)
    if pre is not None:
        p2 = pre.reshape(1, xws[0][0].shape[1])
        in_specs.append(pl.BlockSpec((1, p2.shape[1]), lambda i: (0, 0)))
        args.append(p2)
    nxw = len(xws)
    nadd = len(adds)

    def body(*refs):
        out_ref = refs[-1]
        k = 2 * nxw + nadd
        b_ref = refs[k] if bias is not None else None
        p_ref = refs[k + (1 if bias is not None else 0)] if pre is not None else None
        acc = None
        for i in range(nxw):
            xv = refs[2 * i][...]
            if p_ref is not None:
                xv = jnp.maximum(xv + p_ref[...], 0.0)
            t = jnp.dot(xv, refs[2 * i + 1][...], preferred_element_type=jnp.float32)
            acc = t if acc is None else acc + t
        for i in range(nadd):
            acc = acc + refs[2 * nxw + i][...]
        if b_ref is not None:
            acc = acc + b_ref[...]
        if relu:
            acc = jnp.maximum(acc, 0.0)
        out_ref[...] = acc

    return pl.pallas_call(
        body,
        grid=(grid,),
        in_specs=in_specs,
        out_specs=pl.BlockSpec((block, hout), lambda i: (i, 0)),
        out_shape=jax.ShapeDtypeStruct((n, hout), jnp.float32),
    )(*args)


def _h0_max(nm3):
    def body(x_ref, o_ref):
        o_ref[...] = jnp.max(x_ref[...], axis=1)

    return pl.pallas_call(
        body,
        out_shape=jax.ShapeDtypeStruct((N_MOLS, H), jnp.float32),
    )(nm3)


def _gru_dir(gi3, h0, whhT, bhh, reverse):
    T, Bp = gi3.shape[0], gi3.shape[1]

    def body(gi_ref, h0_ref, w_ref, b_ref, out_ref, h_ref):
        t = pl.program_id(0)

        @pl.when(t == 0)
        def _():
            h_ref[...] = h0_ref[...]

        h = h_ref[...]
        gh = jnp.dot(h, w_ref[...], preferred_element_type=jnp.float32) + b_ref[...]
        gi = gi_ref[0]
        r = jax.nn.sigmoid(gi[:, :H] + gh[:, :H])
        z = jax.nn.sigmoid(gi[:, H:2 * H] + gh[:, H:2 * H])
        nn = jnp.tanh(gi[:, 2 * H:] + r * gh[:, 2 * H:])
        hn = (1.0 - z) * nn + z * h
        h_ref[...] = hn
        out_ref[0] = hn

    if reverse:
        idx = lambda t: (T - 1 - t, 0, 0)
    else:
        idx = lambda t: (t, 0, 0)
    return pl.pallas_call(
        body,
        grid=(T,),
        in_specs=[
            pl.BlockSpec((1, Bp, 3 * H), idx),
            pl.BlockSpec((Bp, H), lambda t: (0, 0)),
            pl.BlockSpec((H, 3 * H), lambda t: (0, 0)),
            pl.BlockSpec((1, 3 * H), lambda t: (0, 0)),
        ],
        out_specs=pl.BlockSpec((1, Bp, H), idx),
        out_shape=jax.ShapeDtypeStruct((T, Bp, H), jnp.float32),
        scratch_shapes=[pltpu.VMEM((Bp, H), jnp.float32)],
    )(gi3, h0, whhT, bhh.reshape(1, 3 * H))


def kernel(f_atoms, f_bonds, a2b, b2a, b2revb, a_scope, W_i_atom, W_i_bond,
           W_h_0, W_h_1, lr_W, W_o, b_o, gru_bias, W_ih_f, W_hh_f, b_ih_f,
           b_hh_f, W_ih_r, W_hh_r, b_ih_r, b_hh_r):
    fa_p = _pad_rows(f_atoms, NA_PAD)
    ia = _mm([(fa_p, W_i_atom.T)], relu=True)
    ib = _mm([(f_bonds, W_i_bond.T)], relu=True)

    ma = ia
    mb = ib
    Whs = [W_h_0, W_h_1]
    for d in range(DEPTH - 1):
        nei = mb[a2b]
        aggv = nei.sum(axis=1) * nei.max(axis=1)
        ma = ma + _pad_rows(aggv, NA_PAD)
        pre = ma[b2a] - mb[b2revb]
        mb = _mm([(pre, Whs[d].T)], adds=(ib,), relu=True)

    nei = mb[a2b]
    aggf = _pad_rows(nei.sum(axis=1) * nei.max(axis=1), NA_PAD)

    cat = jnp.concatenate([aggf, ma, ia], axis=1)
    node = _mm([(cat, lr_W.T)])

    node_seq = node[1:1 + N_MOLS * MOL_SIZE]
    nm3 = node_seq.reshape(N_MOLS, MOL_SIZE, H)
    h0 = _h0_max(nm3)
    Bp = 128
    h0p = _pad_rows(h0, Bp)
    xs_t = jnp.pad(nm3.transpose(1, 0, 2), ((0, 0), (0, Bp - N_MOLS), (0, 0)))
    xs_flat = xs_t.reshape(MOL_SIZE * Bp, H)
    gif = _mm([(xs_flat, W_ih_f.T)], bias=b_ih_f, pre=gru_bias).reshape(MOL_SIZE, Bp, 3 * H)
    gib = _mm([(xs_flat, W_ih_r.T)], bias=b_ih_r, pre=gru_bias).reshape(MOL_SIZE, Bp, 3 * H)
    fwd = _gru_dir(gif, h0p, W_hh_f.T, b_hh_f, reverse=False)
    bwd = _gru_dir(gib, h0p, W_hh_r.T, b_hh_r, reverse=True)
    fwd_mol = fwd[:, :N_MOLS].transpose(1, 0, 2).reshape(N_MOLS * MOL_SIZE, H)
    bwd_mol = bwd[:, :N_MOLS].transpose(1, 0, 2).reshape(N_MOLS * MOL_SIZE, H)

    msg0 = jnp.maximum(node[0:1] + gru_bias[None, :], 0.0)
    fwd_full = _pad_rows(jnp.concatenate([msg0, fwd_mol], axis=0), NA_PAD)
    bwd_full = _pad_rows(jnp.concatenate([msg0, bwd_mol], axis=0), NA_PAD)

    out = _mm([(fwd_full, W_o[:, :H].T), (bwd_full, W_o[:, H:].T)],
              bias=b_o, relu=True)
    return out[:N_ATOMS]


# speedup vs baseline: 1.1720x; 1.1720x over previous
import functools

import jax
import jax.numpy as jnp
from jax import lax
from jax.experimental import pallas as pl
from jax.experimental.pallas import tpu as pltpu
from jax.experimental.pallas import tpu_sc as plsc

H = 128
ATOM_D = 128
BOND_D = 144
N_ATOMS = 10001
N_BONDS = 320000
MAX_NB = 32
N_MOLS = 100
MOL_SIZE = 100
DEPTH = 3

NA_PAD = 10240
NB_PAD = 327680


NC, NS = 2, 16
NW = NC * NS


def _sc_mesh():
    return plsc.VectorSubcoreMesh(core_axis_name="c", subcore_axis_name="s")


def _sc_agg(mb, a2b_r):
    APW = NA_PAD // NW
    NBLK = APW // 4
    IDXR = a2b_r.shape[0] // NW

    @functools.partial(
        pl.kernel,
        out_type=jax.ShapeDtypeStruct((NA_PAD, H), jnp.float32),
        mesh=_sc_mesh(),
        scratch_types=[
            pltpu.VMEM((IDXR, 128), jnp.int32),
            pltpu.VMEM((APW, H), jnp.float32),
            pltpu.VMEM((128, H), jnp.float32),
            pltpu.VMEM((128, H), jnp.float32),
            pltpu.SemaphoreType.DMA,
            pltpu.SemaphoreType.DMA,
        ],
    )
    def k(mb_hbm, a2b_hbm, out_hbm, idx_v, out_v, rows0, rows1, sem0, sem1):
        wid = lax.axis_index("s") * NC + lax.axis_index("c")
        pltpu.sync_copy(a2b_hbm.at[pl.ds(wid * IDXR, IDXR)], idx_v)

        def compute_block(j, rows):
            def atom_body(a, _):
                base = a * 32
                r0 = [rows[base, pl.ds(c * 16, 16)] for c in range(8)]

                def nb(n, car):
                    res = []
                    for c in range(8):
                        r = rows[base + n, pl.ds(c * 16, 16)]
                        res.append(car[c] + r)
                    for c in range(8):
                        r = rows[base + n, pl.ds(c * 16, 16)]
                        res.append(jnp.maximum(car[8 + c], r))
                    return tuple(res)

                fin = lax.fori_loop(1, 32, nb, tuple(r0) + tuple(r0))
                row = j * 4 + a
                for c in range(8):
                    out_v[row, pl.ds(c * 16, 16)] = fin[c] * fin[8 + c]
                return 0

            lax.fori_loop(0, 4, atom_body, 0)

        pltpu.async_copy(mb_hbm.at[idx_v.at[0]], rows0, sem0)

        def body2(jj, _):
            j0 = 2 * jj
            pltpu.async_copy(mb_hbm.at[idx_v.at[j0 + 1]], rows1, sem1)
            pltpu.make_async_copy(mb_hbm.at[idx_v.at[j0]], rows0, sem0).wait()
            compute_block(j0, rows0)

            @pl.when(j0 + 2 < NBLK)
            def _():
                pltpu.async_copy(mb_hbm.at[idx_v.at[j0 + 2]], rows0, sem0)

            pltpu.make_async_copy(mb_hbm.at[idx_v.at[j0 + 1]], rows1, sem1).wait()
            compute_block(j0 + 1, rows1)
            return 0

        lax.fori_loop(0, NBLK // 2, body2, 0)
        pltpu.sync_copy(out_v, out_hbm.at[pl.ds(wid * APW, APW)])

    return k(mb, a2b_r)


def _sc_pre(ma, mb, b2a_r, b2revb_r):
    BPW = NB_PAD // NW
    NBLK = BPW // 128

    @functools.partial(
        pl.kernel,
        out_type=jax.ShapeDtypeStruct((NB_PAD, H), jnp.float32),
        mesh=_sc_mesh(),
        scratch_types=[
            pltpu.VMEM((NBLK, 128), jnp.int32),
            pltpu.VMEM((NBLK, 128), jnp.int32),
            pltpu.VMEM((128, H), jnp.float32),
            pltpu.VMEM((128, H), jnp.float32),
            pltpu.VMEM((128, H), jnp.float32),
            pltpu.VMEM((128, H), jnp.float32),
            pltpu.VMEM((128, H), jnp.float32),
            pltpu.SemaphoreType.DMA,
            pltpu.SemaphoreType.DMA,
            pltpu.SemaphoreType.DMA,
            pltpu.SemaphoreType.DMA,
        ],
    )
    def k(ma_hbm, mb_hbm, b2a_hbm, b2revb_hbm, out_hbm,
          ia_v, ib_v, a0, b0, a1, b1, ob, sa0, sb0, sa1, sb1):
        wid = lax.axis_index("s") * NC + lax.axis_index("c")
        pltpu.sync_copy(b2a_hbm.at[pl.ds(wid * NBLK, NBLK)], ia_v)
        pltpu.sync_copy(b2revb_hbm.at[pl.ds(wid * NBLK, NBLK)], ib_v)

        def start(j, abuf, bbuf, asem, bsem):
            pltpu.async_copy(ma_hbm.at[ia_v.at[j]], abuf, asem)
            pltpu.async_copy(mb_hbm.at[ib_v.at[j]], bbuf, bsem)

        def wait(j, abuf, bbuf, asem, bsem):
            pltpu.make_async_copy(ma_hbm.at[ia_v.at[j]], abuf, asem).wait()
            pltpu.make_async_copy(mb_hbm.at[ib_v.at[j]], bbuf, bsem).wait()

        def compute(j, abuf, bbuf):
            def row_body(r, _):
                for c in range(8):
                    sl = pl.ds(c * 16, 16)
                    ob[r, sl] = abuf[r, sl] - bbuf[r, sl]
                return 0

            lax.fori_loop(0, 128, row_body, 0)
            pltpu.sync_copy(ob, out_hbm.at[pl.ds(wid * BPW + j * 128, 128)])

        start(0, a0, b0, sa0, sb0)

        def body2(jj, _):
            j0 = 2 * jj
            start(j0 + 1, a1, b1, sa1, sb1)
            wait(j0, a0, b0, sa0, sb0)
            compute(j0, a0, b0)

            @pl.when(j0 + 2 < NBLK)
            def _():
                start(j0 + 2, a0, b0, sa0, sb0)

            wait(j0 + 1, a1, b1, sa1, sb1)
            compute(j0 + 1, a1, b1)
            return 0

        lax.fori_loop(0, NBLK // 2, body2, 0)

    return k(ma, mb, b2a_r, b2revb_r)


def _pad_rows(x, n):
    return jnp.pad(x, ((0, n - x.shape[0]),) + ((0, 0),) * (x.ndim - 1))


def _mm(xws, adds=(), bias=None, relu=False, pre=None, block=512, rows=None):
    n = xws[0][0].shape[0]
    rows = n if rows is None else rows
    grid = rows // block
    hout = xws[0][1].shape[1]
    in_specs = []
    args = []
    for x, w in xws:
        in_specs.append(pl.BlockSpec((block, x.shape[1]), lambda i: (i, 0)))
        in_specs.append(pl.BlockSpec(w.shape, lambda i: (0, 0)))
        args += [x, w]
    for a in adds:
        in_specs.append(pl.BlockSpec((block, a.shape[1]), lambda i: (i, 0)))
        args.append(a)
    if bias is not None:
        b2 = bias.reshape(1, hout)
        in_specs.append(pl.BlockSpec((1, hout), lambda i: (0, 0)))
        args.append(b2)
    if pre is not None:
        p2 = pre.reshape(1, xws[0][0].shape[1])
        in_specs.append(pl.BlockSpec((1, p2.shape[1]), lambda i: (0, 0)))
        args.append(p2)
    nxw = len(xws)
    nadd = len(adds)

    def body(*refs):
        out_ref = refs[-1]
        k = 2 * nxw + nadd
        b_ref = refs[k] if bias is not None else None
        p_ref = refs[k + (1 if bias is not None else 0)] if pre is not None else None
        acc = None
        for i in range(nxw):
            xv = refs[2 * i][...]
            if p_ref is not None:
                xv = jnp.maximum(xv + p_ref[...], 0.0)
            t = jnp.dot(xv, refs[2 * i + 1][...], preferred_element_type=jnp.float32)
            acc = t if acc is None else acc + t
        for i in range(nadd):
            acc = acc + refs[2 * nxw + i][...]
        if b_ref is not None:
            acc = acc + b_ref[...]
        if relu:
            acc = jnp.maximum(acc, 0.0)
        out_ref[...] = acc

    return pl.pallas_call(
        body,
        grid=(grid,),
        in_specs=in_specs,
        out_specs=pl.BlockSpec((block, hout), lambda i: (i, 0)),
        out_shape=jax.ShapeDtypeStruct((n, hout), jnp.float32),
    )(*args)


def _h0_max(nm3):
    def body(x_ref, o_ref):
        o_ref[...] = jnp.max(x_ref[...], axis=1)

    return pl.pallas_call(
        body,
        out_shape=jax.ShapeDtypeStruct((N_MOLS, H), jnp.float32),
    )(nm3)


def _gru_dir(gi3, h0, whhT, bhh, reverse):
    T, Bp = gi3.shape[0], gi3.shape[1]

    def body(gi_ref, h0_ref, w_ref, b_ref, out_ref, h_ref):
        t = pl.program_id(0)

        @pl.when(t == 0)
        def _():
            h_ref[...] = h0_ref[...]

        h = h_ref[...]
        gh = jnp.dot(h, w_ref[...], preferred_element_type=jnp.float32) + b_ref[...]
        gi = gi_ref[0]
        r = jax.nn.sigmoid(gi[:, :H] + gh[:, :H])
        z = jax.nn.sigmoid(gi[:, H:2 * H] + gh[:, H:2 * H])
        nn = jnp.tanh(gi[:, 2 * H:] + r * gh[:, 2 * H:])
        hn = (1.0 - z) * nn + z * h
        h_ref[...] = hn
        out_ref[0] = hn

    if reverse:
        idx = lambda t: (T - 1 - t, 0, 0)
    else:
        idx = lambda t: (t, 0, 0)
    return pl.pallas_call(
        body,
        grid=(T,),
        in_specs=[
            pl.BlockSpec((1, Bp, 3 * H), idx),
            pl.BlockSpec((Bp, H), lambda t: (0, 0)),
            pl.BlockSpec((H, 3 * H), lambda t: (0, 0)),
            pl.BlockSpec((1, 3 * H), lambda t: (0, 0)),
        ],
        out_specs=pl.BlockSpec((1, Bp, H), idx),
        out_shape=jax.ShapeDtypeStruct((T, Bp, H), jnp.float32),
        scratch_shapes=[pltpu.VMEM((Bp, H), jnp.float32)],
    )(gi3, h0, whhT, bhh.reshape(1, 3 * H))


def kernel(f_atoms, f_bonds, a2b, b2a, b2revb, a_scope, W_i_atom, W_i_bond,
           W_h_0, W_h_1, lr_W, W_o, b_o, gru_bias, W_ih_f, W_hh_f, b_ih_f,
           b_hh_f, W_ih_r, W_hh_r, b_ih_r, b_hh_r):
    a2b_r = _pad_rows(a2b.astype(jnp.int32), NA_PAD).reshape(-1, 128)
    b2a_r = jnp.pad(b2a.astype(jnp.int32), (0, NB_PAD - N_BONDS)).reshape(-1, 128)
    b2revb_r = jnp.pad(b2revb.astype(jnp.int32), (0, NB_PAD - N_BONDS)).reshape(-1, 128)

    fa_p = _pad_rows(f_atoms, NA_PAD)
    ia = _mm([(fa_p, W_i_atom.T)], relu=True)
    ib = _mm([(f_bonds, W_i_bond.T)], relu=True)

    ma = ia
    mb = ib
    Whs = [W_h_0, W_h_1]
    for d in range(DEPTH - 1):
        ma = ma + _sc_agg(mb, a2b_r)
        pre = _sc_pre(ma, mb, b2a_r, b2revb_r)
        mb = _mm([(pre, Whs[d].T)], adds=(ib,), relu=True, rows=N_BONDS)

    aggf = _sc_agg(mb, a2b_r)

    cat = jnp.concatenate([aggf, ma, ia], axis=1)
    node = _mm([(cat, lr_W.T)])

    node_seq = node[1:1 + N_MOLS * MOL_SIZE]
    nm3 = node_seq.reshape(N_MOLS, MOL_SIZE, H)
    h0 = _h0_max(nm3)
    Bp = 128
    h0p = _pad_rows(h0, Bp)
    xs_t = jnp.pad(nm3.transpose(1, 0, 2), ((0, 0), (0, Bp - N_MOLS), (0, 0)))
    xs_flat = xs_t.reshape(MOL_SIZE * Bp, H)
    gif = _mm([(xs_flat, W_ih_f.T)], bias=b_ih_f, pre=gru_bias).reshape(MOL_SIZE, Bp, 3 * H)
    gib = _mm([(xs_flat, W_ih_r.T)], bias=b_ih_r, pre=gru_bias).reshape(MOL_SIZE, Bp, 3 * H)
    fwd = _gru_dir(gif, h0p, W_hh_f.T, b_hh_f, reverse=False)
    bwd = _gru_dir(gib, h0p, W_hh_r.T, b_hh_r, reverse=True)
    fwd_mol = fwd[:, :N_MOLS].transpose(1, 0, 2).reshape(N_MOLS * MOL_SIZE, H)
    bwd_mol = bwd[:, :N_MOLS].transpose(1, 0, 2).reshape(N_MOLS * MOL_SIZE, H)

    msg0 = jnp.maximum(node[0:1] + gru_bias[None, :], 0.0)
    fwd_full = _pad_rows(jnp.concatenate([msg0, fwd_mol], axis=0), NA_PAD)
    bwd_full = _pad_rows(jnp.concatenate([msg0, bwd_mol], axis=0), NA_PAD)

    msg = jnp.concatenate([fwd_full, bwd_full], axis=1)
    out = _mm([(msg, W_o.T)], bias=b_o, relu=True)
    return out[:N_ATOMS]
